# baseline (speedup 1.0000x reference)
import functools

import jax
import jax.numpy as jnp
from jax import lax
from jax.experimental import pallas as pl
from jax.experimental.pallas import tpu as pltpu
from jax.experimental.pallas import tpu_sc as plsc

DIM = 128
GAMMA = 12.0
EPSILON = 2.0
PI = 3.141592653589793
EMB_RANGE = (GAMMA + EPSILON) / DIM
PHASE_SCALE = PI / EMB_RANGE

NC, NS = 2, 16
NW = NC * NS
CHUNK = 128


def _make_sc_gather(B, ED, n_idx):
    b_per_w = B // NW
    n_chunks = b_per_w // CHUNK
    mesh = plsc.VectorSubcoreMesh(core_axis_name="c", subcore_axis_name="s")

    @functools.partial(
        pl.kernel,
        out_type=[jax.ShapeDtypeStruct((B, ED), jnp.float32)] * n_idx,
        mesh=mesh,
        scratch_types=(
            [pltpu.VMEM((n_chunks, CHUNK), jnp.int32)] * n_idx +
            [pltpu.VMEM((CHUNK, ED), jnp.float32)] * 6 +
            [pltpu.SemaphoreType.DMA] * 12
        ),
    )
    def sc_gather(*refs):
        idx_h = refs[:n_idx]
        tab_h = refs[n_idx]
        outs = refs[n_idx + 1: 2 * n_idx + 1]
        idx_v = refs[2 * n_idx + 1: 3 * n_idx + 1]
        bufs = refs[3 * n_idx + 1: 3 * n_idx + 7]
        gsems = refs[3 * n_idx + 7: 3 * n_idx + 13]
        wsems = refs[3 * n_idx + 13: 3 * n_idx + 19]
        wid = lax.axis_index("s") * NC + lax.axis_index("c")
        base = wid * b_per_w
        for i in range(n_idx):
            pltpu.sync_copy(idx_h[i].at[wid], idx_v[i])

        jobs = []
        for j in range(n_chunks):
            off = base + j * CHUNK
            for i in range(n_idx):
                jobs.append((idx_v[i].at[j], outs[i].at[pl.ds(off, CHUNK)]))

        n = len(jobs)
        NB = 6
        AHEAD = min(4, n)

        def fire_gather(k):
            return pltpu.async_copy(tab_h.at[jobs[k][0]], bufs[k % NB],
                                    gsems[k % NB])

        gcaps = [None] * n
        wcaps = [None] * n
        for j in range(AHEAD):
            gcaps[j] = fire_gather(j)
        w_waited = 0
        for k in range(n):
            gcaps[k].wait()
            wcaps[k] = pltpu.async_copy(bufs[k % NB], jobs[k][1],
                                        wsems[k % NB])
            if k + AHEAD < n:
                prev_w = k + AHEAD - NB
                if prev_w >= 0:
                    wcaps[prev_w].wait()
                    w_waited = prev_w + 1
                gcaps[k + AHEAD] = fire_gather(k + AHEAD)
        for k in range(w_waited, n):
            wcaps[k].wait()

    return sc_gather


def _rel_pad_body(rt_ref, out_ref):
    x = rt_ref[...]
    q = x * (PHASE_SCALE / (2.0 * PI))
    f = q - jnp.round(q)
    ft = jnp.transpose(f, (1, 0))
    out_ref[...] = jnp.concatenate([ft, jnp.zeros_like(ft)], axis=1)


_SIN_C = (6.283088486325916, -41.333249157502294, 81.40011884071671,
          -74.67607214660832, 33.168492067387334)
_COS_C = (0.9999994436793983, -19.739034372931126, 64.93061336990448,
          -85.2959709615383, 58.91255532441487, -21.283021593005525)


def _tc_score_body(head_ref, rel_ref, tail_ref, out_ref):
    h = head_ref[...]
    t = tail_ref[...]
    rp = rel_ref[...]

    def roll(x):
        return pltpu.roll(x, 64, 1)

    lane = lax.broadcasted_iota(jnp.int32, h.shape, 1)
    lo = lane < (h.shape[1] // 2)

    h_sw = roll(h)
    t_sw = roll(t)
    ht = h * t
    a = ht + h_sw * t_sw
    x = h_sw * t
    bp = x - roll(x)
    ab = jnp.where(lo, a, bp)
    hh = h * h + t * t
    s2 = hh + roll(hh)

    f = jnp.where(lo, rp, roll(rp))
    u = f * f
    sp = _SIN_C[4]
    for c in _SIN_C[3::-1]:
        sp = sp * u + c
    sp = sp * f
    cp = _COS_C[5]
    for c in _COS_C[4::-1]:
        cp = cp * u + c
    cs = jnp.where(lo, cp, sp)

    p = cs * ab
    r2 = p + roll(p)
    d = jnp.maximum(s2 - 2.0 * r2, 0.0)
    mag = jnp.sqrt(d)
    out_ref[...] = GAMMA - 0.5 * jnp.sum(mag, axis=1)


def kernel(sample, ent_emb, rel_emb):
    B = sample.shape[0]
    ED = ent_emb.shape[1]
    NSPLIT = 2
    Bh = B // NSPLIT
    n_chunks = Bh // (NW * CHUNK)

    idx = sample.astype(jnp.int32)

    def build_relp():
        NREL = rel_emb.shape[0]
        RD = rel_emb.shape[1]
        CB = 16384
        rt = jnp.transpose(rel_emb)
        return pl.pallas_call(
            _rel_pad_body,
            grid=((NREL + CB - 1) // CB,),
            in_specs=[pl.BlockSpec((RD, CB), lambda i: (0, i))],
            out_specs=pl.BlockSpec((CB, 2 * RD), lambda i: (i, 0)),
            out_shape=jax.ShapeDtypeStruct((NREL, 2 * RD), jnp.float32),
        )(rt)

    sc_ent = _make_sc_gather(Bh, ED, 2)
    sc_rel = _make_sc_gather(Bh, ED, 1)

    BLK = 512
    tc_score = pl.pallas_call(
        _tc_score_body,
        grid=(Bh // BLK,),
        in_specs=[
            pl.BlockSpec((BLK, ED), lambda i: (i, 0)),
            pl.BlockSpec((BLK, ED), lambda i: (i, 0)),
            pl.BlockSpec((BLK, ED), lambda i: (i, 0)),
        ],
        out_specs=pl.BlockSpec((BLK,), lambda i: (i,)),
        out_shape=jax.ShapeDtypeStruct((Bh,), jnp.float32),
    )

    ent_parts = []
    rel_idx = []
    for s in range(NSPLIT):
        sl = idx[s * Bh:(s + 1) * Bh]
        hidx = sl[:, 0].reshape(NW, n_chunks, CHUNK)
        tidx = sl[:, 2].reshape(NW, n_chunks, CHUNK)
        rel_idx.append(sl[:, 1].reshape(NW, n_chunks, CHUNK))
        ent_parts.append(sc_ent(hidx, tidx, ent_emb))

    relp = build_relp()

    scores = []
    for s in range(NSPLIT):
        head, tail = ent_parts[s]
        (relr,) = sc_rel(rel_idx[s], relp)
        scores.append(tc_score(head, relr, tail))
    return jnp.concatenate(scores, axis=0).reshape(B, 1)


# speedup vs baseline: 1.0360x; 1.0360x over previous
import functools

import jax
import jax.numpy as jnp
from jax import lax
from jax.experimental import pallas as pl
from jax.experimental.pallas import tpu as pltpu
from jax.experimental.pallas import tpu_sc as plsc

DIM = 128
GAMMA = 12.0
EPSILON = 2.0
PI = 3.141592653589793
EMB_RANGE = (GAMMA + EPSILON) / DIM
PHASE_SCALE = PI / EMB_RANGE

NC, NS = 2, 16
NW = NC * NS
CHUNK = 128


def _make_sc_gather(B, ED, n_idx):
    b_per_w = B // NW
    n_chunks = b_per_w // CHUNK
    mesh = plsc.VectorSubcoreMesh(core_axis_name="c", subcore_axis_name="s")

    @functools.partial(
        pl.kernel,
        out_type=[jax.ShapeDtypeStruct((B, ED), jnp.float32)] * n_idx,
        mesh=mesh,
        scratch_types=(
            [pltpu.VMEM((n_chunks, CHUNK), jnp.int32)] * n_idx +
            [pltpu.VMEM((CHUNK, ED), jnp.float32)] * 6 +
            [pltpu.SemaphoreType.DMA] * 12
        ),
    )
    def sc_gather(*refs):
        idx_h = refs[:n_idx]
        tab_h = refs[n_idx]
        outs = refs[n_idx + 1: 2 * n_idx + 1]
        idx_v = refs[2 * n_idx + 1: 3 * n_idx + 1]
        bufs = refs[3 * n_idx + 1: 3 * n_idx + 7]
        gsems = refs[3 * n_idx + 7: 3 * n_idx + 13]
        wsems = refs[3 * n_idx + 13: 3 * n_idx + 19]
        wid = lax.axis_index("s") * NC + lax.axis_index("c")
        base = wid * b_per_w
        for i in range(n_idx):
            pltpu.sync_copy(idx_h[i].at[wid], idx_v[i])

        jobs = []
        for j in range(n_chunks):
            off = base + j * CHUNK
            for i in range(n_idx):
                jobs.append((idx_v[i].at[j], outs[i].at[pl.ds(off, CHUNK)]))

        n = len(jobs)
        NB = 6
        AHEAD = min(4, n)

        def fire_gather(k):
            return pltpu.async_copy(tab_h.at[jobs[k][0]], bufs[k % NB],
                                    gsems[k % NB])

        gcaps = [None] * n
        wcaps = [None] * n
        for j in range(AHEAD):
            gcaps[j] = fire_gather(j)
        w_waited = 0
        for k in range(n):
            gcaps[k].wait()
            wcaps[k] = pltpu.async_copy(bufs[k % NB], jobs[k][1],
                                        wsems[k % NB])
            if k + AHEAD < n:
                prev_w = k + AHEAD - NB
                if prev_w >= 0:
                    wcaps[prev_w].wait()
                    w_waited = prev_w + 1
                gcaps[k + AHEAD] = fire_gather(k + AHEAD)
        for k in range(w_waited, n):
            wcaps[k].wait()

    return sc_gather


def _rel_pad_body(rt_ref, out_ref):
    x = rt_ref[...]
    q = x * (PHASE_SCALE / (2.0 * PI))
    f = q - jnp.round(q)
    ft = jnp.transpose(f, (1, 0))
    out_ref[...] = jnp.concatenate([ft, jnp.zeros_like(ft)], axis=1)


_SIN_C = (6.283088486325916, -41.333249157502294, 81.40011884071671,
          -74.67607214660832, 33.168492067387334)
_COS_C = (0.9999994436793983, -19.739034372931126, 64.93061336990448,
          -85.2959709615383, 58.91255532441487, -21.283021593005525)


def _tc_score_body(head_ref, rel_ref, tail_ref, out_ref):
    h = head_ref[...]
    t = tail_ref[...]
    rp = rel_ref[...]

    def roll(x):
        return pltpu.roll(x, 64, 1)

    lane = lax.broadcasted_iota(jnp.int32, h.shape, 1)
    lo = lane < (h.shape[1] // 2)

    h_sw = roll(h)
    t_sw = roll(t)
    ht = h * t
    a = ht + h_sw * t_sw
    x = h_sw * t
    bp = x - roll(x)
    ab = jnp.where(lo, a, bp)
    hh = h * h + t * t
    s2 = hh + roll(hh)

    f = jnp.where(lo, rp, roll(rp))
    u = f * f
    sp = _SIN_C[4]
    for c in _SIN_C[3::-1]:
        sp = sp * u + c
    sp = sp * f
    cp = _COS_C[5]
    for c in _COS_C[4::-1]:
        cp = cp * u + c
    cs = jnp.where(lo, cp, sp)

    p = cs * ab
    r2 = p + roll(p)
    d = jnp.maximum(s2 - 2.0 * r2, 0.0)
    mag = jnp.sqrt(d)
    out_ref[...] = GAMMA - 0.5 * jnp.sum(mag, axis=1)


def kernel(sample, ent_emb, rel_emb):
    B = sample.shape[0]
    ED = ent_emb.shape[1]
    NSPLIT = 2
    Bh = B // NSPLIT
    n_chunks = Bh // (NW * CHUNK)

    idx = sample.astype(jnp.int32)

    def build_relp():
        NREL = rel_emb.shape[0]
        RD = rel_emb.shape[1]
        CB = 16384
        rt = jnp.transpose(rel_emb)
        return pl.pallas_call(
            _rel_pad_body,
            grid=((NREL + CB - 1) // CB,),
            in_specs=[pl.BlockSpec((RD, CB), lambda i: (0, i))],
            out_specs=pl.BlockSpec((CB, 2 * RD), lambda i: (i, 0)),
            out_shape=jax.ShapeDtypeStruct((NREL, 2 * RD), jnp.float32),
        )(rt)

    sc_ent = _make_sc_gather(Bh, ED, 2)
    sc_rel = _make_sc_gather(Bh, ED, 1)

    BLK = 1024
    tc_score = pl.pallas_call(
        _tc_score_body,
        grid=(Bh // BLK,),
        in_specs=[
            pl.BlockSpec((BLK, ED), lambda i: (i, 0)),
            pl.BlockSpec((BLK, ED), lambda i: (i, 0)),
            pl.BlockSpec((BLK, ED), lambda i: (i, 0)),
        ],
        out_specs=pl.BlockSpec((BLK,), lambda i: (i,)),
        out_shape=jax.ShapeDtypeStruct((Bh,), jnp.float32),
    )

    ent_parts = []
    rel_idx = []
    for s in range(NSPLIT):
        sl = idx[s * Bh:(s + 1) * Bh]
        hidx = sl[:, 0].reshape(NW, n_chunks, CHUNK)
        tidx = sl[:, 2].reshape(NW, n_chunks, CHUNK)
        rel_idx.append(sl[:, 1].reshape(NW, n_chunks, CHUNK))
        ent_parts.append(sc_ent(hidx, tidx, ent_emb))

    relp = build_relp()

    scores = []
    for s in range(NSPLIT):
        head, tail = ent_parts[s]
        (relr,) = sc_rel(rel_idx[s], relp)
        scores.append(tc_score(head, relr, tail))
    return jnp.concatenate(scores, axis=0).reshape(B, 1)


# speedup vs baseline: 1.0488x; 1.0124x over previous
import functools

import jax
import jax.numpy as jnp
from jax import lax
from jax.experimental import pallas as pl
from jax.experimental.pallas import tpu as pltpu
from jax.experimental.pallas import tpu_sc as plsc

DIM = 128
GAMMA = 12.0
EPSILON = 2.0
PI = 3.141592653589793
EMB_RANGE = (GAMMA + EPSILON) / DIM
PHASE_SCALE = PI / EMB_RANGE

NC, NS = 2, 16
NW = NC * NS
CHUNK = 128


def _make_sc_gather(B, ED, n_idx):
    b_per_w = B // NW
    n_chunks = b_per_w // CHUNK
    mesh = plsc.VectorSubcoreMesh(core_axis_name="c", subcore_axis_name="s")

    @functools.partial(
        pl.kernel,
        out_type=[jax.ShapeDtypeStruct((B, ED), jnp.float32)] * n_idx,
        mesh=mesh,
        scratch_types=(
            [pltpu.VMEM((n_chunks, CHUNK), jnp.int32)] * n_idx +
            [pltpu.VMEM((CHUNK, ED), jnp.float32)] * 6 +
            [pltpu.SemaphoreType.DMA] * 12
        ),
    )
    def sc_gather(*refs):
        idx_h = refs[:n_idx]
        tab_h = refs[n_idx]
        outs = refs[n_idx + 1: 2 * n_idx + 1]
        idx_v = refs[2 * n_idx + 1: 3 * n_idx + 1]
        bufs = refs[3 * n_idx + 1: 3 * n_idx + 7]
        gsems = refs[3 * n_idx + 7: 3 * n_idx + 13]
        wsems = refs[3 * n_idx + 13: 3 * n_idx + 19]
        wid = lax.axis_index("s") * NC + lax.axis_index("c")
        base = wid * b_per_w
        for i in range(n_idx):
            pltpu.sync_copy(idx_h[i].at[wid], idx_v[i])

        jobs = []
        for j in range(n_chunks):
            off = base + j * CHUNK
            for i in range(n_idx):
                jobs.append((idx_v[i].at[j], outs[i].at[pl.ds(off, CHUNK)]))

        n = len(jobs)
        NB = 6
        AHEAD = min(4, n)

        def fire_gather(k):
            return pltpu.async_copy(tab_h.at[jobs[k][0]], bufs[k % NB],
                                    gsems[k % NB])

        gcaps = [None] * n
        wcaps = [None] * n
        for j in range(AHEAD):
            gcaps[j] = fire_gather(j)
        w_waited = 0
        for k in range(n):
            gcaps[k].wait()
            wcaps[k] = pltpu.async_copy(bufs[k % NB], jobs[k][1],
                                        wsems[k % NB])
            if k + AHEAD < n:
                prev_w = k + AHEAD - NB
                if prev_w >= 0:
                    wcaps[prev_w].wait()
                    w_waited = prev_w + 1
                gcaps[k + AHEAD] = fire_gather(k + AHEAD)
        for k in range(w_waited, n):
            wcaps[k].wait()

    return sc_gather


def _rel_pad_body(rt_ref, out_ref):
    x = rt_ref[...]
    q = x * (PHASE_SCALE / (2.0 * PI))
    f = q - jnp.round(q)
    ft = jnp.transpose(f, (1, 0))
    out_ref[...] = jnp.concatenate([ft, jnp.zeros_like(ft)], axis=1)


_SIN_C = (6.283088486325916, -41.333249157502294, 81.40011884071671,
          -74.67607214660832, 33.168492067387334)
_COS_C = (0.9999994436793983, -19.739034372931126, 64.93061336990448,
          -85.2959709615383, 58.91255532441487, -21.283021593005525)


def _tc_score_body(head_ref, rel_ref, tail_ref, out_ref):
    h = head_ref[...]
    t = tail_ref[...]
    rp = rel_ref[...]

    def roll(x):
        return pltpu.roll(x, 64, 1)

    lane = lax.broadcasted_iota(jnp.int32, h.shape, 1)
    lo = lane < (h.shape[1] // 2)

    h_sw = roll(h)
    t_sw = roll(t)
    ht = h * t
    a = ht + h_sw * t_sw
    x = h_sw * t
    bp = x - roll(x)
    ab = jnp.where(lo, a, bp)
    hh = h * h + t * t
    s2 = hh + roll(hh)

    f = jnp.where(lo, rp, roll(rp))
    u = f * f
    sp = _SIN_C[4]
    for c in _SIN_C[3::-1]:
        sp = sp * u + c
    sp = sp * f
    cp = _COS_C[5]
    for c in _COS_C[4::-1]:
        cp = cp * u + c
    cs = jnp.where(lo, cp, sp)

    p = cs * ab
    r2 = p + roll(p)
    d = jnp.maximum(s2 - 2.0 * r2, 0.0)
    mag = jnp.sqrt(d)
    out_ref[...] = GAMMA - 0.5 * jnp.sum(mag, axis=1)


def kernel(sample, ent_emb, rel_emb):
    B = sample.shape[0]
    ED = ent_emb.shape[1]
    NSPLIT = 2
    Bh = B // NSPLIT
    n_chunks = Bh // (NW * CHUNK)

    idx = sample.astype(jnp.int32)

    def build_relp():
        NREL = rel_emb.shape[0]
        RD = rel_emb.shape[1]
        CB = 32768
        rt = jnp.transpose(rel_emb)
        return pl.pallas_call(
            _rel_pad_body,
            grid=((NREL + CB - 1) // CB,),
            in_specs=[pl.BlockSpec((RD, CB), lambda i: (0, i))],
            out_specs=pl.BlockSpec((CB, 2 * RD), lambda i: (i, 0)),
            out_shape=jax.ShapeDtypeStruct((NREL, 2 * RD), jnp.float32),
        )(rt)

    sc_ent = _make_sc_gather(Bh, ED, 2)
    sc_rel = _make_sc_gather(Bh, ED, 1)

    BLK = 1024
    tc_score = pl.pallas_call(
        _tc_score_body,
        grid=(Bh // BLK,),
        in_specs=[
            pl.BlockSpec((BLK, ED), lambda i: (i, 0)),
            pl.BlockSpec((BLK, ED), lambda i: (i, 0)),
            pl.BlockSpec((BLK, ED), lambda i: (i, 0)),
        ],
        out_specs=pl.BlockSpec((BLK,), lambda i: (i,)),
        out_shape=jax.ShapeDtypeStruct((Bh,), jnp.float32),
    )

    ent_parts = []
    rel_idx = []
    for s in range(NSPLIT):
        sl = idx[s * Bh:(s + 1) * Bh]
        hidx = sl[:, 0].reshape(NW, n_chunks, CHUNK)
        tidx = sl[:, 2].reshape(NW, n_chunks, CHUNK)
        rel_idx.append(sl[:, 1].reshape(NW, n_chunks, CHUNK))
        ent_parts.append(sc_ent(hidx, tidx, ent_emb))

    relp = build_relp()

    scores = []
    for s in range(NSPLIT):
        head, tail = ent_parts[s]
        (relr,) = sc_rel(rel_idx[s], relp)
        scores.append(tc_score(head, relr, tail))
    return jnp.concatenate(scores, axis=0).reshape(B, 1)
